# baseline (speedup 1.0000x reference)
import functools

import jax
import jax.numpy as jnp
from jax import lax
from jax.experimental import pallas as pl
from jax.experimental.pallas import tpu as pltpu
from jax.experimental.pallas import tpu_sc as plsc

N = 883
NP = 896
E = 14128
NW = 32
RPW = 28
ROWW = RPW * NP


def _sc_body(edge_hbm, zeros_hbm, out_hbm, rows_v, cols_v, cmat_v):
    c = lax.axis_index("c")
    s = lax.axis_index("s")
    wid = s * 2 + c
    base = wid * RPW

    pltpu.sync_copy(edge_hbm.at[0], rows_v)
    pltpu.sync_copy(edge_hbm.at[1], cols_v)
    pltpu.sync_copy(zeros_hbm, cmat_v)

    ones = jnp.ones((16,), jnp.float32)

    def one_chunk(off):
        r = rows_v[pl.ds(off, 16)]
        cc = cols_v[pl.ds(off, 16)]
        m = (r >= base) & (r < base + RPW) & (r != cc)
        flat = jnp.where(m, (r - base) * NP + cc, 0)
        plsc.addupdate_scatter(cmat_v, [flat], ones, mask=m)

    nchunk = E // 16
    unroll = 4
    main = nchunk // unroll

    def edge_body(i, carry):
        for u in range(unroll):
            one_chunk((i * unroll + u) * 16)
        return carry

    lax.fori_loop(0, main, edge_body, 0)
    for ch in range(main * unroll, nchunk):
        one_chunk(ch * 16)

    pltpu.sync_copy(cmat_v, out_hbm.at[wid])


@functools.cache
def _sc_count_fn():
    return pl.kernel(
        _sc_body,
        out_type=jax.ShapeDtypeStruct((NW, ROWW), jnp.float32),
        mesh=plsc.VectorSubcoreMesh(core_axis_name="c", subcore_axis_name="s"),
        compiler_params=pltpu.CompilerParams(needs_layout_passes=False),
        scratch_types=[
            pltpu.VMEM((E,), jnp.int32),
            pltpu.VMEM((E,), jnp.int32),
            pltpu.VMEM((ROWW,), jnp.float32),
        ],
    )


def _dot(a, b):
    return jnp.dot(a, b, preferred_element_type=jnp.float32)


def _dot_nt(a, b):
    return lax.dot_general(a, b, (((1,), (1,)), ((), ())),
                           preferred_element_type=jnp.float32)


def _dot_tn(a, b):
    return lax.dot_general(a, b, (((0,), (0,)), ((), ())),
                           preferred_element_type=jnp.float32)


def _sig(z):
    return 1.0 / (1.0 + jnp.exp(-z))


def _softmax0(z):
    mx = jnp.max(z, axis=0, keepdims=True)
    e = jnp.exp(z - mx)
    return e / jnp.sum(e, axis=0, keepdims=True)


def _layer_norm(y, g, b):
    mu = jnp.mean(y, axis=1, keepdims=True)
    var = jnp.mean((y - mu) * (y - mu), axis=1, keepdims=True)
    return (y - mu) * lax.rsqrt(var + 1e-5) * g + b


def _tc_body(cmat_ref, x_ref,
             u1r0, u2r0, u3s0, be0, ve0, w1c0, w2r0, w3s0, bs0, vs0,
             w_all, cb0t, tw0m, tb0t, rw0m, rb0t, g0t, bb0t,
             u1r1, u2p1, be1, ve1, w1r1, w2tile, bs1, vs1,
             u3bd, w3bd, bdcw1_0, bdcw1_1, bdcw1_2, cb1t, tw1m, tb1t,
             bd_rw1, rb1t, g1t, bb1t, bd_mean,
             fwt, fb, out_ref):
    rmask = lax.broadcasted_iota(jnp.int32, (NP, 1), 0) < N
    eye = (lax.broadcasted_iota(jnp.int32, (NP, NP), 0)
           == lax.broadcasted_iota(jnp.int32, (NP, NP), 1))

    def softmax0_masked(z):
        zm = jnp.where(rmask, z, -1e30)
        mx = jnp.max(zm, axis=0, keepdims=True)
        e = jnp.where(rmask, jnp.exp(zm - mx), 0.0)
        return e / jnp.sum(e, axis=0, keepdims=True)

    def diag_col(smat):
        return jnp.sum(jnp.where(eye, smat, 0.0), axis=1, keepdims=True)

    cmat = cmat_ref[...]
    deg = jnp.sum(cmat, axis=1, keepdims=True)
    dis = jnp.where(deg > 0.0, lax.rsqrt(jnp.maximum(deg, 1e-30)), 0.0)
    nmat = -(_dot_nt(dis, dis) * cmat)

    def layer_norm_blockwise(y, g, b):
        mu = _dot(y, bd_mean[...])
        ctr = y - mu
        var = _dot(ctr * ctr, bd_mean[...])
        return ctr * lax.rsqrt(var + 1e-5) * g + b

    for b in range(2):
        x0 = x_ref[b]

        u1x = _dot(u1r0[...], x0)
        rhs_t = u3s0[...] * x0
        v = _dot(u2r0[...], rhs_t)
        p = _dot_tn(u1x, v)
        et = _softmax0(_dot(ve0[...], _sig(p + be0[...])))
        xt = _dot(x0, et)

        sw1 = _dot(xt, w1c0[...])
        lhs_s = _dot(sw1, w2r0[...])
        rhs_s = w3s0[...] * xt
        spre = _dot_nt(lhs_s, rhs_s) + bs0[...]
        smat = softmax0_masked(_dot(vs0[...], _sig(spre)))

        dcol = diag_col(smat)
        tx0 = dcol * x0
        tx1 = _dot(nmat * smat, tx0)
        tx2 = 2.0 * _dot(nmat, tx1) - tx0
        txc = jnp.concatenate([tx0, tx1, tx2], axis=1)

        xh0c = jnp.maximum(_dot(txc, w_all[...]) + cb0t[...], 0.0)
        y0 = jnp.maximum(_dot(xh0c, tw0m[...]) + tb0t[...]
                         + _dot(x0, rw0m[...]) + rb0t[...], 0.0)
        xlc = layer_norm_blockwise(y0, g0t[...], bb0t[...])

        u1row = _dot(u1r1[...], xlc)
        rhsm = _dot(xlc, u3bd[...])
        m1 = _dot(u2p1[...], rhsm)
        p1 = jnp.concatenate(
            [_dot(lax.slice(u1row, (0, 64 * t), (1, 64 * t + 64)), m1)
             for t in range(6)], axis=0)
        et1 = _softmax0(_dot(ve1[...], _sig(p1 + be1[...])))

        acoef = _dot_nt(et1, w1r1[...])
        atile = jnp.concatenate(
            [jnp.broadcast_to(lax.slice(acoef, (t, 0), (t + 1, 1)), (64, 1))
             for t in range(6)], axis=0)
        lhs1 = _dot(xlc, atile * w2tile[...])
        zm = _dot(xlc, w3bd[...])
        rhs1 = _dot(zm, et1)
        spre1 = _dot_nt(lhs1, rhs1) + bs1[...]
        smat1 = softmax0_masked(_dot(vs1[...], _sig(spre1)))

        dcol1 = diag_col(smat1)
        tx0c = dcol1 * xlc
        tx1c = _dot(nmat * smat1, tx0c)
        tx2c = 2.0 * _dot(nmat, tx1c) - tx0c
        xh1c = jnp.maximum(_dot(tx0c, bdcw1_0[...]) + _dot(tx1c, bdcw1_1[...])
                           + _dot(tx2c, bdcw1_2[...]) + cb1t[...], 0.0)
        y1 = jnp.maximum(_dot(xh1c, tw1m[...]) + tb1t[...]
                         + _dot(xlc, bd_rw1[...]) + rb1t[...], 0.0)
        x2c = layer_norm_blockwise(y1, g1t[...], bb1t[...])

        ob = jnp.maximum(_dot(x2c, fwt[...]) + fb[...], 0.0)
        out_ref[b] = ob


def _padn(a, axis):
    pad = [(0, 0)] * a.ndim
    pad[axis] = (0, NP - N)
    return jnp.pad(a, pad)


def _prep_operands(x, params):
    p0 = params['block0']
    p1 = params['block1']
    xp = _padn(x[:, :, 0, :], 1)

    bi66 = jnp.arange(384)[:, None] // 64
    bj66 = jnp.arange(384)[None, :] // 64
    bd_mean = jnp.where(bi66 == bj66, 1.0 / 64.0, 0.0)
    cw1 = p1['cheb_w']
    bdcw1 = [jnp.where(bi66 == bj66, jnp.tile(cw1[k], (6, 6)), 0.0)
             for k in range(3)]
    twt1 = jnp.transpose(p1['time_w'][:, :, 0, :], (2, 1, 0))
    tw1m = sum(jnp.where(bi66 - bj66 == k - 1, jnp.tile(twt1[k], (6, 6)), 0.0)
               for k in range(3))
    rwt1 = jnp.transpose(p1['res_w'][:, :, 0, 0])
    bd_rw1 = jnp.where(bi66 == bj66, jnp.tile(rwt1, (6, 6)), 0.0)

    cw0 = p0['cheb_w'][:, 0, :]
    i12x768 = jnp.arange(12)[:, None]
    j12x768 = jnp.arange(768)[None, :] // 64
    w_all = jnp.concatenate(
        [jnp.where(i12x768 == j12x768, jnp.tile(cw0[k:k + 1, :], (1, 12)), 0.0)
         for k in range(3)], axis=0)
    twt0 = jnp.transpose(p0['time_w'][:, :, 0, :], (2, 1, 0))
    bi126 = jnp.arange(768)[:, None] // 64
    bj126 = jnp.arange(384)[None, :] // 64
    tw0m = sum(jnp.where(bi126 - 2 * bj126 == k - 1,
                         jnp.tile(twt0[k], (12, 6)), 0.0)
               for k in range(3))
    rw0 = p0['res_w'][:, 0, 0, 0][None, :]
    i12x384 = jnp.arange(12)[:, None]
    j12x384 = jnp.arange(384)[None, :] // 64
    rw0m = jnp.where(i12x384 == 2 * j12x384, jnp.tile(rw0, (1, 6)), 0.0)

    i384x6 = jnp.arange(384)[:, None] // 64
    j384x6 = jnp.arange(6)[None, :]
    u3bd = jnp.where(i384x6 == j384x6, jnp.tile(p1['U3'][:, None], (6, 1)), 0.0)
    w3bd = jnp.where(i384x6 == j384x6, jnp.tile(p1['W3'][:, None], (6, 1)), 0.0)

    def t6(a):
        return jnp.tile(a, (1, 6))

    ops = [
        xp,
        _padn(p0['U1'][None, :], 1),
        _padn(p0['U2'], 1),
        p0['U3'][None, :],
        p0['be'][0],
        p0['Ve'],
        p0['W1'][:, None],
        p0['W2'],
        p0['W3'][None, :],
        _padn(_padn(p0['bs'][0], 0), 1),
        _padn(_padn(p0['Vs'], 0), 1),
        w_all,
        jnp.tile(p0['cheb_b'][None, :], (1, 12)),
        tw0m,
        t6(p0['time_b'][None, :]),
        rw0m,
        t6(p0['res_b'][None, :]),
        t6(p0['ln_g'][None, :]),
        t6(p0['ln_b'][None, :]),
        _padn(p1['U1'][None, :], 1),
        _padn(p1['U2'], 1),
        p1['be'][0],
        p1['Ve'],
        p1['W1'][None, :],
        jnp.tile(p1['W2'], (6, 1)),
        _padn(_padn(p1['bs'][0], 0), 1),
        _padn(_padn(p1['Vs'], 0), 1),
        u3bd,
        w3bd,
        bdcw1[0], bdcw1[1], bdcw1[2],
        t6(p1['cheb_b'][None, :]),
        tw1m,
        t6(p1['time_b'][None, :]),
        bd_rw1,
        t6(p1['res_b'][None, :]),
        t6(p1['ln_g'][None, :]),
        t6(p1['ln_b'][None, :]),
        bd_mean,
        jnp.transpose(params['final_w'].reshape(12, 6 * 64)),
        params['final_b'][None, :],
    ]
    return ops


def _tc_model(cmat, ops):
    return pl.pallas_call(
        _tc_body,
        out_shape=jax.ShapeDtypeStruct((2, NP, 12), jnp.float32),
    )(cmat, *ops)


def kernel(x, edge_index, params):
    zeros = jnp.zeros((ROWW,), jnp.float32)
    cmat = _sc_count_fn()(edge_index, zeros).reshape(NP, NP)
    out = _tc_model(cmat, _prep_operands(x, params))
    return out[:, :N, :]


# speedup vs baseline: 1.2757x; 1.2757x over previous
import functools

import jax
import jax.numpy as jnp
from jax import lax
from jax.experimental import pallas as pl
from jax.experimental.pallas import tpu as pltpu
from jax.experimental.pallas import tpu_sc as plsc

N = 883
NP = 896
E = 14128
NW = 32
RPW = 28
ROWW = RPW * NP


def _sc_body(edge_hbm, zeros_hbm, out_hbm, rows_v, cols_v, cmat_v):
    c = lax.axis_index("c")
    s = lax.axis_index("s")
    wid = s * 2 + c
    base = wid * RPW

    pltpu.sync_copy(edge_hbm.at[0], rows_v)
    pltpu.sync_copy(edge_hbm.at[1], cols_v)
    pltpu.sync_copy(zeros_hbm, cmat_v)

    ones = jnp.ones((16,), jnp.float32)

    def one_chunk(off):
        r = rows_v[pl.ds(off, 16)]
        cc = cols_v[pl.ds(off, 16)]
        m = (r >= base) & (r < base + RPW) & (r != cc)
        flat = jnp.where(m, (r - base) * NP + cc, 0)
        plsc.addupdate_scatter(cmat_v, [flat], ones, mask=m)

    nchunk = E // 16
    unroll = 4
    main = nchunk // unroll

    def edge_body(i, carry):
        for u in range(unroll):
            one_chunk((i * unroll + u) * 16)
        return carry

    lax.fori_loop(0, main, edge_body, 0)
    for ch in range(main * unroll, nchunk):
        one_chunk(ch * 16)

    pltpu.sync_copy(cmat_v, out_hbm.at[wid])


@functools.cache
def _sc_count_fn():
    return pl.kernel(
        _sc_body,
        out_type=jax.ShapeDtypeStruct((NW, ROWW), jnp.float32),
        mesh=plsc.VectorSubcoreMesh(core_axis_name="c", subcore_axis_name="s"),
        compiler_params=pltpu.CompilerParams(needs_layout_passes=False),
        scratch_types=[
            pltpu.VMEM((E,), jnp.int32),
            pltpu.VMEM((E,), jnp.int32),
            pltpu.VMEM((ROWW,), jnp.float32),
        ],
    )


def _dot(a, b):
    return jnp.dot(a, b, preferred_element_type=jnp.float32)


def _dot_nt(a, b):
    return lax.dot_general(a, b, (((1,), (1,)), ((), ())),
                           preferred_element_type=jnp.float32)


def _dot_tn(a, b):
    return lax.dot_general(a, b, (((0,), (0,)), ((), ())),
                           preferred_element_type=jnp.float32)


def _sig(z):
    return 1.0 / (1.0 + jnp.exp(-z))


def _softmax0(z):
    mx = jnp.max(z, axis=0, keepdims=True)
    e = jnp.exp(z - mx)
    return e / jnp.sum(e, axis=0, keepdims=True)


def _layer_norm(y, g, b):
    mu = jnp.mean(y, axis=1, keepdims=True)
    var = jnp.mean((y - mu) * (y - mu), axis=1, keepdims=True)
    return (y - mu) * lax.rsqrt(var + 1e-5) * g + b


def _tc_body(cmat_ref, x_ref,
             u1r0, u2r0, u3s0, be0, ve0, w1c0, w2r0, w3s0, bs0, vs0,
             cw0, cb0, twt0, tb0, rw0, rb0, g0, bb0,
             u1r1, u2p1, u3c1, be1, ve1, w1r1, w2b1, w3c1, bs1, vs1,
             cw1, cb1, twt1, tb1, rwt1, rb1, g1, bb1,
             fwt, fb, out_ref):
    rmask = lax.broadcasted_iota(jnp.int32, (NP, 1), 0) < N
    eye = (lax.broadcasted_iota(jnp.int32, (NP, NP), 0)
           == lax.broadcasted_iota(jnp.int32, (NP, NP), 1))

    def softmax0_masked(z):
        zm = jnp.where(rmask, z, -1e30)
        mx = jnp.max(zm, axis=0, keepdims=True)
        e = jnp.where(rmask, jnp.exp(zm - mx), 0.0)
        return e / jnp.sum(e, axis=0, keepdims=True)

    def diag_col(smat):
        return jnp.sum(jnp.where(eye, smat, 0.0), axis=1, keepdims=True)

    cmat = cmat_ref[...]
    deg = jnp.sum(cmat, axis=1, keepdims=True)
    dis = jnp.where(deg > 0.0, lax.rsqrt(jnp.maximum(deg, 1e-30)), 0.0)
    nmat = -(_dot_nt(dis, dis) * cmat)

    def _tile(w, nr, nc):
        row = jnp.concatenate([w] * nc, axis=1)
        return jnp.concatenate([row] * nr, axis=0) if nr > 1 else row

    bi66 = lax.broadcasted_iota(jnp.int32, (384, 384), 0) // 64
    bj66 = lax.broadcasted_iota(jnp.int32, (384, 384), 1) // 64
    zero66 = jnp.zeros((384, 384), jnp.float32)
    bd_mean = jnp.where(bi66 == bj66, 1.0 / 64.0, 0.0)
    bd_cw1 = [jnp.where(bi66 == bj66, _tile(cw1[k], 6, 6), zero66)
              for k in range(3)]
    tw1m = sum(jnp.where(bi66 - bj66 == k - 1, _tile(twt1[k], 6, 6), zero66)
               for k in range(3))
    bd_rw1 = jnp.where(bi66 == bj66, _tile(rwt1[...], 6, 6), zero66)

    i12x768 = lax.broadcasted_iota(jnp.int32, (12, 768), 0)
    j12x768 = lax.broadcasted_iota(jnp.int32, (12, 768), 1) // 64
    w_all = jnp.concatenate(
        [jnp.where(i12x768 == j12x768, _tile(cw0[k:k + 1, :], 1, 12), 0.0)
         for k in range(3)], axis=0)
    bi126 = lax.broadcasted_iota(jnp.int32, (768, 384), 0) // 64
    bj126 = lax.broadcasted_iota(jnp.int32, (768, 384), 1) // 64
    zero126 = jnp.zeros((768, 384), jnp.float32)
    tw0m = sum(jnp.where(bi126 - 2 * bj126 == k - 1, _tile(twt0[k], 12, 6),
                         zero126) for k in range(3))
    i12x384 = lax.broadcasted_iota(jnp.int32, (12, 384), 0)
    j12x384 = lax.broadcasted_iota(jnp.int32, (12, 384), 1) // 64
    rw0m = jnp.where(i12x384 == 2 * j12x384, _tile(rw0[...], 1, 6), 0.0)
    i384x6 = lax.broadcasted_iota(jnp.int32, (384, 6), 0) // 64
    j384x6 = lax.broadcasted_iota(jnp.int32, (384, 6), 1)
    u3bd = jnp.where(i384x6 == j384x6, _tile(u3c1[...], 6, 1), 0.0)
    w3bd = jnp.where(i384x6 == j384x6, _tile(w3c1[...], 6, 1), 0.0)
    w2tile = _tile(w2b1[...], 6, 1)

    cb0t = _tile(cb0[...], 1, 12)
    tb0t = _tile(tb0[...], 1, 6)
    rb0t = _tile(rb0[...], 1, 6)
    g0t = _tile(g0[...], 1, 6)
    bb0t = _tile(bb0[...], 1, 6)
    cb1t = _tile(cb1[...], 1, 6)
    tb1t = _tile(tb1[...], 1, 6)
    rb1t = _tile(rb1[...], 1, 6)
    g1t = _tile(g1[...], 1, 6)
    bb1t = _tile(bb1[...], 1, 6)

    def layer_norm_blockwise(y, g, b):
        mu = _dot(y, bd_mean)
        ctr = y - mu
        var = _dot(ctr * ctr, bd_mean)
        return ctr * lax.rsqrt(var + 1e-5) * g + b

    for b in range(2):
        x0 = x_ref[b]

        u1x = _dot(u1r0[...], x0)
        rhs_t = u3s0[...] * x0
        v = _dot(u2r0[...], rhs_t)
        p = _dot_tn(u1x, v)
        et = _softmax0(_dot(ve0[...], _sig(p + be0[...])))
        xt = _dot(x0, et)

        sw1 = _dot(xt, w1c0[...])
        lhs_s = _dot(sw1, w2r0[...])
        rhs_s = w3s0[...] * xt
        spre = _dot_nt(lhs_s, rhs_s) + bs0[...]
        smat = softmax0_masked(_dot(vs0[...], _sig(spre)))

        dcol = diag_col(smat)
        tx0 = dcol * x0
        tx1 = _dot(nmat * smat, tx0)
        tx2 = 2.0 * _dot(nmat, tx1) - tx0
        txc = jnp.concatenate([tx0, tx1, tx2], axis=1)

        xh0c = jnp.maximum(_dot(txc, w_all) + cb0t, 0.0)
        y0 = jnp.maximum(_dot(xh0c, tw0m) + tb0t
                         + _dot(x0, rw0m) + rb0t, 0.0)
        xlc = layer_norm_blockwise(y0, g0t, bb0t)

        u1row = _dot(u1r1[...], xlc)
        rhsm = _dot(xlc, u3bd)
        m1 = _dot(u2p1[...], rhsm)
        p1 = jnp.concatenate(
            [_dot(lax.slice(u1row, (0, 64 * t), (1, 64 * t + 64)), m1)
             for t in range(6)], axis=0)
        et1 = _softmax0(_dot(ve1[...], _sig(p1 + be1[...])))

        acoef = _dot_nt(et1, w1r1[...])
        atile = jnp.concatenate(
            [jnp.broadcast_to(lax.slice(acoef, (t, 0), (t + 1, 1)), (64, 1))
             for t in range(6)], axis=0)
        lhs1 = _dot(xlc, atile * w2tile)
        zm = _dot(xlc, w3bd)
        rhs1 = _dot(zm, et1)
        spre1 = _dot_nt(lhs1, rhs1) + bs1[...]
        smat1 = softmax0_masked(_dot(vs1[...], _sig(spre1)))

        dcol1 = diag_col(smat1)
        tx0c = dcol1 * xlc
        tx1c = _dot(nmat * smat1, tx0c)
        tx2c = 2.0 * _dot(nmat, tx1c) - tx0c
        xh1c = jnp.maximum(_dot(tx0c, bd_cw1[0]) + _dot(tx1c, bd_cw1[1])
                           + _dot(tx2c, bd_cw1[2]) + cb1t, 0.0)
        y1 = jnp.maximum(_dot(xh1c, tw1m) + tb1t
                         + _dot(xlc, bd_rw1) + rb1t, 0.0)
        x2c = layer_norm_blockwise(y1, g1t, bb1t)

        ob = jnp.maximum(_dot(x2c, fwt[...]) + fb[...], 0.0)
        out_ref[b] = ob


def _padn(a, axis):
    pad = [(0, 0)] * a.ndim
    pad[axis] = (0, NP - N)
    return jnp.pad(a, pad)


def _prep_operands(x, params):
    p0 = params['block0']
    p1 = params['block1']
    xp = _padn(x[:, :, 0, :], 1)
    ops = [
        xp,
        _padn(p0['U1'][None, :], 1),
        _padn(p0['U2'], 1),
        p0['U3'][None, :],
        p0['be'][0],
        p0['Ve'],
        p0['W1'][:, None],
        p0['W2'],
        p0['W3'][None, :],
        _padn(_padn(p0['bs'][0], 0), 1),
        _padn(_padn(p0['Vs'], 0), 1),
        p0['cheb_w'][:, 0, :],
        p0['cheb_b'][None, :],
        jnp.transpose(p0['time_w'][:, :, 0, :], (2, 1, 0)),
        p0['time_b'][None, :],
        p0['res_w'][:, 0, 0, 0][None, :],
        p0['res_b'][None, :],
        p0['ln_g'][None, :],
        p0['ln_b'][None, :],
        _padn(p1['U1'][None, :], 1),
        _padn(p1['U2'], 1),
        p1['U3'][:, None],
        p1['be'][0],
        p1['Ve'],
        p1['W1'][None, :],
        p1['W2'],
        p1['W3'][:, None],
        _padn(_padn(p1['bs'][0], 0), 1),
        _padn(_padn(p1['Vs'], 0), 1),
        p1['cheb_w'],
        p1['cheb_b'][None, :],
        jnp.transpose(p1['time_w'][:, :, 0, :], (2, 1, 0)),
        p1['time_b'][None, :],
        jnp.transpose(p1['res_w'][:, :, 0, 0]),
        p1['res_b'][None, :],
        p1['ln_g'][None, :],
        p1['ln_b'][None, :],
        jnp.transpose(params['final_w'].reshape(12, 6 * 64)),
        params['final_b'][None, :],
    ]
    return ops


def _tc_model(cmat, ops):
    return pl.pallas_call(
        _tc_body,
        out_shape=jax.ShapeDtypeStruct((2, NP, 12), jnp.float32),
    )(cmat, *ops)


def kernel(x, edge_index, params):
    zeros = jnp.zeros((ROWW,), jnp.float32)
    cmat = _sc_count_fn()(edge_index, zeros).reshape(NP, NP)
    out = _tc_model(cmat, _prep_operands(x, params))
    return out[:, :N, :]


# speedup vs baseline: 1.2792x; 1.0027x over previous
import functools

import jax
import jax.numpy as jnp
from jax import lax
from jax.experimental import pallas as pl
from jax.experimental.pallas import tpu as pltpu
from jax.experimental.pallas import tpu_sc as plsc

N = 883
NP = 896
E = 14128
NW = 32
RPW = 28
ROWW = RPW * NP


def _sc_body(edge_hbm, zeros_hbm, out_hbm, rows_v, cols_v, cmat_v):
    c = lax.axis_index("c")
    s = lax.axis_index("s")
    wid = s * 2 + c
    base = wid * RPW

    pltpu.sync_copy(edge_hbm.at[0], rows_v)
    pltpu.sync_copy(edge_hbm.at[1], cols_v)
    pltpu.sync_copy(zeros_hbm, cmat_v)

    ones = jnp.ones((16,), jnp.float32)

    def one_chunk(off):
        r = rows_v[pl.ds(off, 16)]
        cc = cols_v[pl.ds(off, 16)]
        m = (r >= base) & (r < base + RPW) & (r != cc)
        flat = jnp.where(m, (r - base) * NP + cc, 0)
        plsc.addupdate_scatter(cmat_v, [flat], ones, mask=m)

    nchunk = E // 16
    unroll = 4
    main = nchunk // unroll

    def edge_body(i, carry):
        for u in range(unroll):
            one_chunk((i * unroll + u) * 16)
        return carry

    lax.fori_loop(0, main, edge_body, 0)
    for ch in range(main * unroll, nchunk):
        one_chunk(ch * 16)

    pltpu.sync_copy(cmat_v, out_hbm.at[wid])


@functools.cache
def _sc_count_fn():
    return pl.kernel(
        _sc_body,
        out_type=jax.ShapeDtypeStruct((NW, ROWW), jnp.float32),
        mesh=plsc.VectorSubcoreMesh(core_axis_name="c", subcore_axis_name="s"),
        compiler_params=pltpu.CompilerParams(needs_layout_passes=False),
        scratch_types=[
            pltpu.VMEM((E,), jnp.int32),
            pltpu.VMEM((E,), jnp.int32),
            pltpu.VMEM((ROWW,), jnp.float32),
        ],
    )


def _dot(a, b):
    return jnp.dot(a, b, preferred_element_type=jnp.float32)


def _dot_nt(a, b):
    return lax.dot_general(a, b, (((1,), (1,)), ((), ())),
                           preferred_element_type=jnp.float32)


def _dot_tn(a, b):
    return lax.dot_general(a, b, (((0,), (0,)), ((), ())),
                           preferred_element_type=jnp.float32)


def _sig(z):
    return 1.0 / (1.0 + jnp.exp(-z))


def _softmax0(z):
    mx = jnp.max(z, axis=0, keepdims=True)
    e = jnp.exp(z - mx)
    return e / jnp.sum(e, axis=0, keepdims=True)


def _layer_norm(y, g, b):
    mu = jnp.mean(y, axis=1, keepdims=True)
    var = jnp.mean((y - mu) * (y - mu), axis=1, keepdims=True)
    return (y - mu) * lax.rsqrt(var + 1e-5) * g + b


def _tc_body(cmat_ref, x_ref,
             u1r0, u2r0, u3s0, be0, ve0, w1c0, w2r0, w3s0, bs0, vs0,
             cw0, cb0, twt0, tb0, rw0, rb0, g0, bb0,
             u1r1, u2p1, u3c1, be1, ve1, w1r1, w2b1, w3c1, bs1, vs1,
             cw1, cb1, twt1, tb1, rwt1, rb1, g1, bb1,
             fwt, fb, out_ref):
    rmask = lax.broadcasted_iota(jnp.int32, (NP, 1), 0) < N
    eye = (lax.broadcasted_iota(jnp.int32, (NP, NP), 0)
           == lax.broadcasted_iota(jnp.int32, (NP, NP), 1))

    def softmax0_masked(z):
        zm = jnp.where(rmask, z, -1e30)
        mx = jnp.max(zm, axis=0, keepdims=True)
        e = jnp.where(rmask, jnp.exp(zm - mx), 0.0)
        return e / jnp.sum(e, axis=0, keepdims=True)

    def diag_col(smat):
        return jnp.sum(jnp.where(eye, smat, 0.0), axis=1, keepdims=True)

    cmat = cmat_ref[...]
    deg = jnp.sum(cmat, axis=1, keepdims=True)
    dis = jnp.where(deg > 0.0, lax.rsqrt(jnp.maximum(deg, 1e-30)), 0.0)
    nmat = -(_dot_nt(dis, dis) * cmat)

    def _tile(w, nr, nc):
        row = jnp.concatenate([w] * nc, axis=1)
        return jnp.concatenate([row] * nr, axis=0) if nr > 1 else row

    bi66 = lax.broadcasted_iota(jnp.int32, (384, 384), 0) // 64
    bj66 = lax.broadcasted_iota(jnp.int32, (384, 384), 1) // 64
    zero66 = jnp.zeros((384, 384), jnp.float32)
    bd_mean = jnp.where(bi66 == bj66, 1.0 / 64.0, 0.0)
    bd_cw1 = [jnp.where(bi66 == bj66, _tile(cw1[k], 6, 6), zero66)
              for k in range(3)]
    tw1m = sum(jnp.where(bi66 - bj66 == k - 1, _tile(twt1[k], 6, 6), zero66)
               for k in range(3))
    bd_rw1 = jnp.where(bi66 == bj66, _tile(rwt1[...], 6, 6), zero66)

    i12x768 = lax.broadcasted_iota(jnp.int32, (12, 768), 0)
    j12x768 = lax.broadcasted_iota(jnp.int32, (12, 768), 1) // 64
    w_all = jnp.concatenate(
        [jnp.where(i12x768 == j12x768, _tile(cw0[k:k + 1, :], 1, 12), 0.0)
         for k in range(3)], axis=0)
    bi126 = lax.broadcasted_iota(jnp.int32, (768, 384), 0) // 64
    bj126 = lax.broadcasted_iota(jnp.int32, (768, 384), 1) // 64
    zero126 = jnp.zeros((768, 384), jnp.float32)
    tw0m = sum(jnp.where(bi126 - 2 * bj126 == k - 1, _tile(twt0[k], 12, 6),
                         zero126) for k in range(3))
    i12x384 = lax.broadcasted_iota(jnp.int32, (12, 384), 0)
    j12x384 = lax.broadcasted_iota(jnp.int32, (12, 384), 1) // 64
    rw0m = jnp.where(i12x384 == 2 * j12x384, _tile(rw0[...], 1, 6), 0.0)
    i384x6 = lax.broadcasted_iota(jnp.int32, (384, 6), 0) // 64
    j384x6 = lax.broadcasted_iota(jnp.int32, (384, 6), 1)
    u3bd = jnp.where(i384x6 == j384x6, _tile(u3c1[...], 6, 1), 0.0)
    w3bd = jnp.where(i384x6 == j384x6, _tile(w3c1[...], 6, 1), 0.0)
    w2tile = _tile(w2b1[...], 6, 1)

    cb0t = _tile(cb0[...], 1, 12)
    tb0t = _tile(tb0[...], 1, 6)
    rb0t = _tile(rb0[...], 1, 6)
    g0t = _tile(g0[...], 1, 6)
    bb0t = _tile(bb0[...], 1, 6)
    cb1t = _tile(cb1[...], 1, 6)
    tb1t = _tile(tb1[...], 1, 6)
    rb1t = _tile(rb1[...], 1, 6)
    g1t = _tile(g1[...], 1, 6)
    bb1t = _tile(bb1[...], 1, 6)

    def layer_norm_blockwise(y, g, b):
        mu = _dot(y, bd_mean)
        ctr = y - mu
        var = _dot(ctr * ctr, bd_mean)
        return ctr * lax.rsqrt(var + 1e-5) * g + b

    for b in range(2):
        x0 = x_ref[b]

        u1x = _dot(u1r0[...], x0)
        rhs_t = u3s0[...] * x0
        v = _dot(u2r0[...], rhs_t)
        p = _dot_tn(u1x, v)
        et = _softmax0(_dot(ve0[...], _sig(p + be0[...])))
        xt = _dot(x0, et)

        sw1 = _dot(xt, w1c0[...])
        lhs_s = _dot(sw1, w2r0[...])
        rhs_s = w3s0[...] * xt
        spre = _dot_nt(lhs_s, rhs_s) + bs0[...]
        smat = softmax0_masked(
            _dot(vs0[...].astype(jnp.bfloat16), _sig(spre).astype(jnp.bfloat16)))

        dcol = diag_col(smat)
        tx0 = dcol * x0
        tx1 = _dot(nmat * smat, tx0)
        tx2 = 2.0 * _dot(nmat, tx1) - tx0
        txc = jnp.concatenate([tx0, tx1, tx2], axis=1)

        xh0c = jnp.maximum(_dot(txc, w_all) + cb0t, 0.0)
        y0 = jnp.maximum(_dot(xh0c, tw0m) + tb0t
                         + _dot(x0, rw0m) + rb0t, 0.0)
        xlc = layer_norm_blockwise(y0, g0t, bb0t)

        u1row = _dot(u1r1[...], xlc)
        rhsm = _dot(xlc, u3bd)
        m1 = _dot(u2p1[...], rhsm)
        p1 = jnp.concatenate(
            [_dot(lax.slice(u1row, (0, 64 * t), (1, 64 * t + 64)), m1)
             for t in range(6)], axis=0)
        et1 = _softmax0(_dot(ve1[...], _sig(p1 + be1[...])))

        acoef = _dot_nt(et1, w1r1[...])
        atile = jnp.concatenate(
            [jnp.broadcast_to(lax.slice(acoef, (t, 0), (t + 1, 1)), (64, 1))
             for t in range(6)], axis=0)
        lhs1 = _dot(xlc, atile * w2tile)
        zm = _dot(xlc, w3bd)
        rhs1 = _dot(zm, et1)
        spre1 = _dot_nt(lhs1, rhs1) + bs1[...]
        smat1 = softmax0_masked(
            _dot(vs1[...].astype(jnp.bfloat16), _sig(spre1).astype(jnp.bfloat16)))

        dcol1 = diag_col(smat1)
        tx0c = dcol1 * xlc
        tx1c = _dot(nmat * smat1, tx0c)
        tx2c = 2.0 * _dot(nmat, tx1c) - tx0c
        xh1c = jnp.maximum(_dot(tx0c, bd_cw1[0]) + _dot(tx1c, bd_cw1[1])
                           + _dot(tx2c, bd_cw1[2]) + cb1t, 0.0)
        y1 = jnp.maximum(_dot(xh1c, tw1m) + tb1t
                         + _dot(xlc, bd_rw1) + rb1t, 0.0)
        x2c = layer_norm_blockwise(y1, g1t, bb1t)

        ob = jnp.maximum(_dot(x2c, fwt[...]) + fb[...], 0.0)
        out_ref[b] = ob


def _padn(a, axis):
    pad = [(0, 0)] * a.ndim
    pad[axis] = (0, NP - N)
    return jnp.pad(a, pad)


def _prep_operands(x, params):
    p0 = params['block0']
    p1 = params['block1']
    xp = _padn(x[:, :, 0, :], 1)
    ops = [
        xp,
        _padn(p0['U1'][None, :], 1),
        _padn(p0['U2'], 1),
        p0['U3'][None, :],
        p0['be'][0],
        p0['Ve'],
        p0['W1'][:, None],
        p0['W2'],
        p0['W3'][None, :],
        _padn(_padn(p0['bs'][0], 0), 1),
        _padn(_padn(p0['Vs'], 0), 1),
        p0['cheb_w'][:, 0, :],
        p0['cheb_b'][None, :],
        jnp.transpose(p0['time_w'][:, :, 0, :], (2, 1, 0)),
        p0['time_b'][None, :],
        p0['res_w'][:, 0, 0, 0][None, :],
        p0['res_b'][None, :],
        p0['ln_g'][None, :],
        p0['ln_b'][None, :],
        _padn(p1['U1'][None, :], 1),
        _padn(p1['U2'], 1),
        p1['U3'][:, None],
        p1['be'][0],
        p1['Ve'],
        p1['W1'][None, :],
        p1['W2'],
        p1['W3'][:, None],
        _padn(_padn(p1['bs'][0], 0), 1),
        _padn(_padn(p1['Vs'], 0), 1),
        p1['cheb_w'],
        p1['cheb_b'][None, :],
        jnp.transpose(p1['time_w'][:, :, 0, :], (2, 1, 0)),
        p1['time_b'][None, :],
        jnp.transpose(p1['res_w'][:, :, 0, 0]),
        p1['res_b'][None, :],
        p1['ln_g'][None, :],
        p1['ln_b'][None, :],
        jnp.transpose(params['final_w'].reshape(12, 6 * 64)),
        params['final_b'][None, :],
    ]
    return ops


def _tc_model(cmat, ops):
    return pl.pallas_call(
        _tc_body,
        out_shape=jax.ShapeDtypeStruct((2, NP, 12), jnp.float32),
    )(cmat, *ops)


def kernel(x, edge_index, params):
    zeros = jnp.zeros((ROWW,), jnp.float32)
    cmat = _sc_count_fn()(edge_index, zeros).reshape(NP, NP)
    out = _tc_model(cmat, _prep_operands(x, params))
    return out[:, :N, :]


# speedup vs baseline: 1.2905x; 1.0088x over previous
import functools

import jax
import jax.numpy as jnp
from jax import lax
from jax.experimental import pallas as pl
from jax.experimental.pallas import tpu as pltpu
from jax.experimental.pallas import tpu_sc as plsc

N = 883
NP = 896
E = 14128
NW = 32
RPW = 28
ROWW = RPW * NP


def _sc_body(edge_hbm, out_hbm, rows_v, cols_v, cmat_v):
    c = lax.axis_index("c")
    s = lax.axis_index("s")
    wid = s * 2 + c
    base = wid * RPW

    pltpu.sync_copy(edge_hbm.at[0], rows_v)
    pltpu.sync_copy(edge_hbm.at[1], cols_v)

    def zero_body(i, carry):
        cmat_v[pl.ds(i * 16, 16)] = jnp.zeros((16,), jnp.float32)
        return carry

    lax.fori_loop(0, ROWW // 16, zero_body, 0)

    ones = jnp.ones((16,), jnp.float32)

    def edge_body(i, carry):
        r = rows_v[pl.ds(i * 16, 16)]
        cc = cols_v[pl.ds(i * 16, 16)]
        m = (r >= base) & (r < base + RPW) & (r != cc)
        flat = jnp.where(m, (r - base) * NP + cc, 0)
        plsc.addupdate_scatter(cmat_v, [flat], ones, mask=m)
        return carry

    lax.fori_loop(0, E // 16, edge_body, 0)

    pltpu.sync_copy(cmat_v, out_hbm.at[wid])


@functools.cache
def _sc_count_fn():
    return pl.kernel(
        _sc_body,
        out_type=jax.ShapeDtypeStruct((NW, ROWW), jnp.float32),
        mesh=plsc.VectorSubcoreMesh(core_axis_name="c", subcore_axis_name="s"),
        compiler_params=pltpu.CompilerParams(needs_layout_passes=False),
        scratch_types=[
            pltpu.VMEM((E,), jnp.int32),
            pltpu.VMEM((E,), jnp.int32),
            pltpu.VMEM((ROWW,), jnp.float32),
        ],
    )


def _dot(a, b):
    return jnp.dot(a, b, preferred_element_type=jnp.float32)


def _dot_nt(a, b):
    return lax.dot_general(a, b, (((1,), (1,)), ((), ())),
                           preferred_element_type=jnp.float32)


def _dot_tn(a, b):
    return lax.dot_general(a, b, (((0,), (0,)), ((), ())),
                           preferred_element_type=jnp.float32)


def _sig(z):
    return 1.0 / (1.0 + jnp.exp(-z))


def _softmax0(z):
    mx = jnp.max(z, axis=0, keepdims=True)
    e = jnp.exp(z - mx)
    return e / jnp.sum(e, axis=0, keepdims=True)


def _layer_norm(y, g, b):
    mu = jnp.mean(y, axis=1, keepdims=True)
    var = jnp.mean((y - mu) * (y - mu), axis=1, keepdims=True)
    return (y - mu) * lax.rsqrt(var + 1e-5) * g + b


def _tc_body(cmat_ref, x_ref,
             u1r0, u2r0, u3s0, be0, ve0, w1c0, w2r0, w3s0, bs0, vs0,
             cw0, cb0, twt0, tb0, rw0, rb0, g0, bb0,
             u1r1, u2p1, u3c1, be1, ve1, w1r1, w2b1, w3c1, bs1, vs1,
             cw1, cb1, twt1, tb1, rwt1, rb1, g1, bb1,
             fwt, fb, out_ref):
    rmask = lax.broadcasted_iota(jnp.int32, (NP, 1), 0) < N
    eye = (lax.broadcasted_iota(jnp.int32, (NP, NP), 0)
           == lax.broadcasted_iota(jnp.int32, (NP, NP), 1))

    def softmax0_masked(z):
        zm = jnp.where(rmask, z, -1e30)
        mx = jnp.max(zm, axis=0, keepdims=True)
        e = jnp.where(rmask, jnp.exp(zm - mx), 0.0)
        return e / jnp.sum(e, axis=0, keepdims=True)

    def diag_col(smat):
        return jnp.sum(jnp.where(eye, smat, 0.0), axis=1, keepdims=True)

    cmat = cmat_ref[...]
    deg = jnp.sum(cmat, axis=1, keepdims=True)
    dis = jnp.where(deg > 0.0, lax.rsqrt(jnp.maximum(deg, 1e-30)), 0.0)
    nmat = -(_dot_nt(dis, dis) * cmat)

    def _tile(w, nr, nc):
        row = jnp.concatenate([w] * nc, axis=1)
        return jnp.concatenate([row] * nr, axis=0) if nr > 1 else row

    bi66 = lax.broadcasted_iota(jnp.int32, (384, 384), 0) // 64
    bj66 = lax.broadcasted_iota(jnp.int32, (384, 384), 1) // 64
    zero66 = jnp.zeros((384, 384), jnp.float32)
    bd_mean = jnp.where(bi66 == bj66, 1.0 / 64.0, 0.0)
    bd_cw1 = [jnp.where(bi66 == bj66, _tile(cw1[k], 6, 6), zero66)
              for k in range(3)]
    tw1m = sum(jnp.where(bi66 - bj66 == k - 1, _tile(twt1[k], 6, 6), zero66)
               for k in range(3))
    bd_rw1 = jnp.where(bi66 == bj66, _tile(rwt1[...], 6, 6), zero66)

    i12x768 = lax.broadcasted_iota(jnp.int32, (12, 768), 0)
    j12x768 = lax.broadcasted_iota(jnp.int32, (12, 768), 1) // 64
    w_all = jnp.concatenate(
        [jnp.where(i12x768 == j12x768, _tile(cw0[k:k + 1, :], 1, 12), 0.0)
         for k in range(3)], axis=0)
    bi126 = lax.broadcasted_iota(jnp.int32, (768, 384), 0) // 64
    bj126 = lax.broadcasted_iota(jnp.int32, (768, 384), 1) // 64
    zero126 = jnp.zeros((768, 384), jnp.float32)
    tw0m = sum(jnp.where(bi126 - 2 * bj126 == k - 1, _tile(twt0[k], 12, 6),
                         zero126) for k in range(3))
    i12x384 = lax.broadcasted_iota(jnp.int32, (12, 384), 0)
    j12x384 = lax.broadcasted_iota(jnp.int32, (12, 384), 1) // 64
    rw0m = jnp.where(i12x384 == 2 * j12x384, _tile(rw0[...], 1, 6), 0.0)
    i384x6 = lax.broadcasted_iota(jnp.int32, (384, 6), 0) // 64
    j384x6 = lax.broadcasted_iota(jnp.int32, (384, 6), 1)
    u3bd = jnp.where(i384x6 == j384x6, _tile(u3c1[...], 6, 1), 0.0)
    w3bd = jnp.where(i384x6 == j384x6, _tile(w3c1[...], 6, 1), 0.0)
    w2tile = _tile(w2b1[...], 6, 1)

    cb0t = _tile(cb0[...], 1, 12)
    tb0t = _tile(tb0[...], 1, 6)
    rb0t = _tile(rb0[...], 1, 6)
    g0t = _tile(g0[...], 1, 6)
    bb0t = _tile(bb0[...], 1, 6)
    cb1t = _tile(cb1[...], 1, 6)
    tb1t = _tile(tb1[...], 1, 6)
    rb1t = _tile(rb1[...], 1, 6)
    g1t = _tile(g1[...], 1, 6)
    bb1t = _tile(bb1[...], 1, 6)

    def layer_norm_blockwise(y, g, b):
        mu = _dot(y, bd_mean)
        ctr = y - mu
        var = _dot(ctr * ctr, bd_mean)
        return ctr * lax.rsqrt(var + 1e-5) * g + b

    for b in range(2):
        x0 = x_ref[b]

        u1x = _dot(u1r0[...], x0)
        rhs_t = u3s0[...] * x0
        v = _dot(u2r0[...], rhs_t)
        p = _dot_tn(u1x, v)
        et = _softmax0(_dot(ve0[...], _sig(p + be0[...])))
        xt = _dot(x0, et)

        sw1 = _dot(xt, w1c0[...])
        lhs_s = _dot(sw1, w2r0[...])
        rhs_s = w3s0[...] * xt
        spre = _dot_nt(lhs_s, rhs_s) + bs0[...]
        smat = softmax0_masked(_dot(vs0[...], _sig(spre)))

        dcol = diag_col(smat)
        tx0 = dcol * x0
        tx1 = _dot(nmat * smat, tx0)
        tx2 = 2.0 * _dot(nmat, tx1) - tx0
        txc = jnp.concatenate([tx0, tx1, tx2], axis=1)

        xh0c = jnp.maximum(_dot(txc, w_all) + cb0t, 0.0)
        y0 = jnp.maximum(_dot(xh0c, tw0m) + tb0t
                         + _dot(x0, rw0m) + rb0t, 0.0)
        xlc = layer_norm_blockwise(y0, g0t, bb0t)

        u1row = _dot(u1r1[...], xlc)
        rhsm = _dot(xlc, u3bd)
        m1 = _dot(u2p1[...], rhsm)
        p1 = jnp.concatenate(
            [_dot(lax.slice(u1row, (0, 64 * t), (1, 64 * t + 64)), m1)
             for t in range(6)], axis=0)
        et1 = _softmax0(_dot(ve1[...], _sig(p1 + be1[...])))

        acoef = _dot_nt(et1, w1r1[...])
        atile = jnp.concatenate(
            [jnp.broadcast_to(lax.slice(acoef, (t, 0), (t + 1, 1)), (64, 1))
             for t in range(6)], axis=0)
        lhs1 = _dot(xlc, atile * w2tile)
        zm = _dot(xlc, w3bd)
        rhs1 = _dot(zm, et1)
        spre1 = _dot_nt(lhs1, rhs1) + bs1[...]
        smat1 = softmax0_masked(_dot(vs1[...], _sig(spre1)))

        dcol1 = diag_col(smat1)
        tx0c = dcol1 * xlc
        tx1c = _dot(nmat * smat1, tx0c)
        tx2c = 2.0 * _dot(nmat, tx1c) - tx0c
        xh1c = jnp.maximum(_dot(tx0c, bd_cw1[0]) + _dot(tx1c, bd_cw1[1])
                           + _dot(tx2c, bd_cw1[2]) + cb1t, 0.0)
        y1 = jnp.maximum(_dot(xh1c, tw1m) + tb1t
                         + _dot(xlc, bd_rw1) + rb1t, 0.0)
        x2c = layer_norm_blockwise(y1, g1t, bb1t)

        ob = jnp.maximum(_dot(x2c, fwt[...]) + fb[...], 0.0)
        out_ref[b] = ob


def _padn(a, axis):
    pad = [(0, 0)] * a.ndim
    pad[axis] = (0, NP - N)
    return jnp.pad(a, pad)


def _prep_operands(x, params):
    p0 = params['block0']
    p1 = params['block1']
    xp = _padn(x[:, :, 0, :], 1)
    ops = [
        xp,
        _padn(p0['U1'][None, :], 1),
        _padn(p0['U2'], 1),
        p0['U3'][None, :],
        p0['be'][0],
        p0['Ve'],
        p0['W1'][:, None],
        p0['W2'],
        p0['W3'][None, :],
        _padn(_padn(p0['bs'][0], 0), 1),
        _padn(_padn(p0['Vs'], 0), 1),
        p0['cheb_w'][:, 0, :],
        p0['cheb_b'][None, :],
        jnp.transpose(p0['time_w'][:, :, 0, :], (2, 1, 0)),
        p0['time_b'][None, :],
        p0['res_w'][:, 0, 0, 0][None, :],
        p0['res_b'][None, :],
        p0['ln_g'][None, :],
        p0['ln_b'][None, :],
        _padn(p1['U1'][None, :], 1),
        _padn(p1['U2'], 1),
        p1['U3'][:, None],
        p1['be'][0],
        p1['Ve'],
        p1['W1'][None, :],
        p1['W2'],
        p1['W3'][:, None],
        _padn(_padn(p1['bs'][0], 0), 1),
        _padn(_padn(p1['Vs'], 0), 1),
        p1['cheb_w'],
        p1['cheb_b'][None, :],
        jnp.transpose(p1['time_w'][:, :, 0, :], (2, 1, 0)),
        p1['time_b'][None, :],
        jnp.transpose(p1['res_w'][:, :, 0, 0]),
        p1['res_b'][None, :],
        p1['ln_g'][None, :],
        p1['ln_b'][None, :],
        jnp.transpose(params['final_w'].reshape(12, 6 * 64)),
        params['final_b'][None, :],
    ]
    return ops


def _tc_model(cmat, ops):
    return pl.pallas_call(
        _tc_body,
        out_shape=jax.ShapeDtypeStruct((2, NP, 12), jnp.float32),
    )(cmat, *ops)


def kernel(x, edge_index, params):
    cmat = _sc_count_fn()(edge_index).reshape(NP, NP)
    out = _tc_model(cmat, _prep_operands(x, params))
    return out[:, :N, :]
